# baseline (speedup 1.0000x reference)
import functools

import jax
import jax.numpy as jnp
from jax import lax
from jax.experimental import pallas as pl
from jax.experimental.pallas import tpu as pltpu
from jax.experimental.pallas import tpu_sc as plsc

N = 10000
E = 320000
D_IN = 128
D = 64

NC = 2
NS = 16
NW = NC * NS
L = 16

NP = 10240
RPT = NP // NS
CHUNK = 128
CH = 80
EP = NW * CH * CHUNK
PAD_IDX = NP - 1
DEGW = 16

BR = 256

_sc_mesh = plsc.VectorSubcoreMesh(
    core_axis_name="c", subcore_axis_name="s", num_cores=NC, num_subcores=NS
)


NBUF = 4
ZR = 128


def _sc_segsum_body(table_hbm, src_hbm, dst_hbm, out_hbm,
                    idx_s, idx_d, rows, acc, zbuf, *sems):
    cid = lax.axis_index("c")
    sid = lax.axis_index("s")
    wid = cid * NS + sid

    gsems = sems[:NBUF]
    ssems = sems[NBUF:]

    pltpu.sync_copy(src_hbm.at[wid], idx_s)
    pltpu.sync_copy(dst_hbm.at[wid], idx_d)

    zero = jnp.zeros((L,), jnp.float32)

    @pl.loop(0, ZR)
    def _zero_rows(i):
        for j in range(D // L):
            zbuf[i, pl.ds(j * L, L)] = zero

    for r in range(RPT // ZR):
        pltpu.sync_copy(zbuf, acc.at[pl.ds(sid * RPT + r * ZR, ZR)])
    plsc.subcore_barrier()

    for b in range(NBUF):
        pltpu.async_copy(table_hbm.at[idx_s.at[b]], rows.at[b], gsems[b])

    @pl.loop(0, CH, step=NBUF)
    def _chunks(ch):
        for b in range(NBUF):
            pltpu.make_async_copy(
                table_hbm.at[idx_s.at[ch + b]], rows.at[b], gsems[b]).wait()
            pltpu.async_copy(rows.at[b], acc.at[idx_d.at[ch + b]], ssems[b],
                             add=True)
        for b in range(NBUF):
            @pl.when(ch + NBUF + b < CH)
            def _():
                pltpu.make_async_copy(
                    rows.at[b], acc.at[idx_d.at[ch + b]], ssems[b]).wait()
                pltpu.async_copy(
                    table_hbm.at[idx_s.at[ch + NBUF + b]], rows.at[b], gsems[b])

    for b in range(NBUF):
        pltpu.make_async_copy(
            rows.at[b], acc.at[idx_d.at[CH - NBUF + b]], ssems[b]).wait()

    plsc.subcore_barrier()
    pltpu.sync_copy(acc.at[pl.ds(sid * RPT, RPT)], out_hbm.at[wid])


def _make_sc_segsum(interpret=False):
    return pl.kernel(
        _sc_segsum_body,
        out_type=jax.ShapeDtypeStruct((NW, RPT, D), jnp.float32),
        mesh=_sc_mesh,
        compiler_params=pltpu.CompilerParams(use_tc_tiling_on_sc=False),
        scratch_types=[
            pltpu.VMEM((CH, CHUNK), jnp.int32),
            pltpu.VMEM((CH, CHUNK), jnp.int32),
            pltpu.VMEM((NBUF, CHUNK, D), jnp.float32),
            pltpu.VMEM_SHARED((NP, D), jnp.float32),
            pltpu.VMEM((ZR, D), jnp.float32),
        ] + [pltpu.SemaphoreType.DMA] * (2 * NBUF),
        interpret=interpret,
    )


_sc_segsum = _make_sc_segsum()


def _sc_degree_body(src_hbm, out_hbm, idx_s, ones_b, acc, zbuf):
    cid = lax.axis_index("c")
    sid = lax.axis_index("s")
    wid = cid * NS + sid

    pltpu.sync_copy(src_hbm.at[wid], idx_s)

    one = jnp.ones((L,), jnp.float32)
    zero = jnp.zeros((L,), jnp.float32)

    @pl.loop(0, CHUNK)
    def _fill_ones(i):
        ones_b[i, :] = one

    @pl.loop(0, RPT)
    def _zero_rows(i):
        zbuf[i, :] = zero

    pltpu.sync_copy(zbuf, acc.at[pl.ds(sid * RPT, RPT)])
    plsc.subcore_barrier()

    @pl.loop(0, CH)
    def _chunks(ch):
        pltpu.sync_copy(ones_b, acc.at[idx_s.at[ch]], add=True)

    plsc.subcore_barrier()
    pltpu.sync_copy(acc.at[pl.ds(sid * RPT, RPT)], out_hbm.at[wid])


def _make_sc_degree(interpret=False):
    return pl.kernel(
        _sc_degree_body,
        out_type=jax.ShapeDtypeStruct((NW, RPT, DEGW), jnp.float32),
        mesh=_sc_mesh,
        compiler_params=pltpu.CompilerParams(use_tc_tiling_on_sc=False),
        scratch_types=[
            pltpu.VMEM((CH, CHUNK), jnp.int32),
            pltpu.VMEM((CHUNK, DEGW), jnp.float32),
            pltpu.VMEM_SHARED((NP, DEGW), jnp.float32),
            pltpu.VMEM((RPT, DEGW), jnp.float32),
        ],
        interpret=interpret,
    )


_sc_degree = _make_sc_degree()


def _dinv_body(g_ref, dinv_ref):
    g = g_ref[...]
    deg = g[0, :, 0:1] + g[1, :, 0:1]
    dinv_ref[...] = jnp.where(deg > 0, lax.rsqrt(jnp.maximum(deg, 1e-12)), 0.0)


def _tc_dinv(degraw):
    return pl.pallas_call(
        _dinv_body,
        grid=(NP // BR,),
        in_specs=[pl.BlockSpec((NC, BR, DEGW), lambda i: (0, i, 0))],
        out_specs=pl.BlockSpec((BR, 1), lambda i: (i, 0)),
        out_shape=jax.ShapeDtypeStruct((NP, 1), jnp.float32),
    )(degraw)


def _dot(a, b):
    return jnp.dot(a, b, preferred_element_type=jnp.float32,
                   precision=lax.Precision.HIGHEST)


def _mm3(h, w_ref, b_ref, d, p_ref, y1_ref, c_ref):
    w0 = w_ref[0]
    w1 = w_ref[1]
    w2 = w_ref[2]
    p_ref[...] = d * _dot(h, w2)
    y1_ref[...] = _dot(h, w1)
    c_ref[...] = _dot(h, w0 - w2) + b_ref[...]


def _k1_body(h_ref, w_ref, b_ref, dinv_ref, p_ref, y1_ref, c_ref):
    _mm3(h_ref[...], w_ref, b_ref, dinv_ref[...], p_ref, y1_ref, c_ref)


def _tc_layer_in(h, W, b, dinv, din):
    out_sds = jax.ShapeDtypeStruct((NP, D), jnp.float32)
    return pl.pallas_call(
        _k1_body,
        grid=(NP // BR,),
        in_specs=[
            pl.BlockSpec((BR, din), lambda i: (i, 0)),
            pl.BlockSpec((3, din, D), lambda i: (0, 0, 0)),
            pl.BlockSpec((1, D), lambda i: (0, 0)),
            pl.BlockSpec((BR, 1), lambda i: (i, 0)),
        ],
        out_specs=[pl.BlockSpec((BR, D), lambda i: (i, 0))] * 3,
        out_shape=[out_sds, out_sds, out_sds],
    )(h, W, b.reshape(1, D), dinv)


def _k13_body(c_ref, r_ref, dinv_ref, w_ref, b_ref, p_ref, y1_ref, c2_ref):
    d = dinv_ref[...]
    h = c_ref[...] - d * (r_ref[0] + r_ref[1])
    h = jnp.maximum(h, 0.0)
    _mm3(h, w_ref, b_ref, d, p_ref, y1_ref, c2_ref)


def _tc_layer_boundary(c, rraw, dinv, W, b):
    out_sds = jax.ShapeDtypeStruct((NP, D), jnp.float32)
    return pl.pallas_call(
        _k13_body,
        grid=(NP // BR,),
        in_specs=[
            pl.BlockSpec((BR, D), lambda i: (i, 0)),
            pl.BlockSpec((NC, BR, D), lambda i: (0, i, 0)),
            pl.BlockSpec((BR, 1), lambda i: (i, 0)),
            pl.BlockSpec((3, D, D), lambda i: (0, 0, 0)),
            pl.BlockSpec((1, D), lambda i: (0, 0)),
        ],
        out_specs=[pl.BlockSpec((BR, D), lambda i: (i, 0))] * 3,
        out_shape=[out_sds, out_sds, out_sds],
    )(c, rraw, dinv, W, b.reshape(1, D))


def _k2_body(z_ref, y1_ref, dinv_ref, u_ref):
    d = dinv_ref[...]
    u_ref[...] = d * y1_ref[...] - (2.0 * d * d) * (z_ref[0] + z_ref[1])


def _tc_mid(zraw, y1, dinv):
    return pl.pallas_call(
        _k2_body,
        grid=(NP // BR,),
        in_specs=[
            pl.BlockSpec((NC, BR, D), lambda i: (0, i, 0)),
            pl.BlockSpec((BR, D), lambda i: (i, 0)),
            pl.BlockSpec((BR, 1), lambda i: (i, 0)),
        ],
        out_specs=pl.BlockSpec((BR, D), lambda i: (i, 0)),
        out_shape=jax.ShapeDtypeStruct((NP, D), jnp.float32),
    )(zraw, y1, dinv)


def _k3_body(c_ref, r_ref, dinv_ref, o_ref):
    o_ref[...] = c_ref[...] - dinv_ref[...] * (r_ref[0] + r_ref[1])


def _tc_final(c, rraw, dinv):
    return pl.pallas_call(
        _k3_body,
        grid=(NP // BR,),
        in_specs=[
            pl.BlockSpec((BR, D), lambda i: (i, 0)),
            pl.BlockSpec((NC, BR, D), lambda i: (0, i, 0)),
            pl.BlockSpec((BR, 1), lambda i: (i, 0)),
        ],
        out_specs=pl.BlockSpec((BR, D), lambda i: (i, 0)),
        out_shape=jax.ShapeDtypeStruct((NP, D), jnp.float32),
    )(c, rraw, dinv)


def _segsum(table, src_t, dst_t):
    raw = _sc_segsum(table, src_t, dst_t)
    return raw.reshape(NC, NP, D)


def kernel(x, edge_index, W1, b1, W2, b2, W3, b3):
    xp = jnp.zeros((NP, D_IN), jnp.float32).at[:N].set(x)
    pad = jnp.full((EP - E,), PAD_IDX, jnp.int32)
    src_t = jnp.concatenate([edge_index[0], pad]).reshape(NW, CH, CHUNK)
    dst_t = jnp.concatenate([edge_index[1], pad]).reshape(NW, CH, CHUNK)

    degraw = _sc_degree(src_t).reshape(NC, NP, DEGW)
    dinv = _tc_dinv(degraw)

    p, y1, c = _tc_layer_in(xp, W1, b1, dinv, D_IN)
    for (W, b) in ((W2, b2), (W3, b3)):
        zraw = _segsum(p, src_t, dst_t)
        u = _tc_mid(zraw, y1, dinv)
        rraw = _segsum(u, src_t, dst_t)
        p, y1, c = _tc_layer_boundary(c, rraw, dinv, W, b)
    zraw = _segsum(p, src_t, dst_t)
    u = _tc_mid(zraw, y1, dinv)
    rraw = _segsum(u, src_t, dst_t)
    out = _tc_final(c, rraw, dinv)
    return out[:N]


# speedup vs baseline: 2.1731x; 2.1731x over previous
import functools

import jax
import jax.numpy as jnp
from jax import lax
from jax.experimental import pallas as pl
from jax.experimental.pallas import tpu as pltpu
from jax.experimental.pallas import tpu_sc as plsc

N = 10000
E = 320000
D_IN = 128
D = 64
DH = D // 2

NC = 2
NS = 16
NW = NC * NS
L = 16

NP = 10240
RPT = NP // NS
CHUNK = 128
CH = 160
EP = NS * CH * CHUNK
PAD_IDX = NP - 1
DEGW = 16
CHD = 80

BR = 256

_sc_mesh = plsc.VectorSubcoreMesh(
    core_axis_name="c", subcore_axis_name="s", num_cores=NC, num_subcores=NS
)


NBUF = 4
ZR = 128


def _sc_segsum_body(table_hbm, src_hbm, dst_hbm, out_hbm,
                    idx_s, idx_d, rows, tbl, acc, zbuf, *sems):
    cid = lax.axis_index("c")
    sid = lax.axis_index("s")
    wid = cid * NS + sid

    pltpu.sync_copy(table_hbm.at[cid, pl.ds(sid * RPT, RPT)],
                    tbl.at[pl.ds(sid * RPT, RPT)])

    pltpu.sync_copy(src_hbm.at[sid], idx_s)
    pltpu.sync_copy(dst_hbm.at[sid], idx_d)

    zero = jnp.zeros((L,), jnp.float32)

    @pl.loop(0, ZR)
    def _zero_rows(i):
        for j in range(DH // L):
            zbuf[i, pl.ds(j * L, L)] = zero

    for r in range(RPT // ZR):
        pltpu.sync_copy(zbuf, acc.at[pl.ds(sid * RPT + r * ZR, ZR)])
    plsc.subcore_barrier()

    for b in range(NBUF):
        pltpu.async_copy(tbl.at[idx_s.at[b]], rows.at[b], sems[b])

    @pl.loop(0, CH, step=NBUF)
    def _chunks(ch):
        for b in range(NBUF):
            pltpu.make_async_copy(
                tbl.at[idx_s.at[ch + b]], rows.at[b], sems[b]).wait()
            pltpu.sync_copy(rows.at[b], acc.at[idx_d.at[ch + b]], add=True)

            @pl.when(ch + NBUF + b < CH)
            def _():
                pltpu.async_copy(
                    tbl.at[idx_s.at[ch + NBUF + b]], rows.at[b], sems[b])

    plsc.subcore_barrier()
    pltpu.sync_copy(acc.at[pl.ds(sid * RPT, RPT)], out_hbm.at[wid])


def _make_sc_segsum(interpret=False):
    return pl.kernel(
        _sc_segsum_body,
        out_type=jax.ShapeDtypeStruct((NW, RPT, DH), jnp.float32),
        mesh=_sc_mesh,
        compiler_params=pltpu.CompilerParams(use_tc_tiling_on_sc=False),
        scratch_types=[
            pltpu.VMEM((CH, CHUNK), jnp.int32),
            pltpu.VMEM((CH, CHUNK), jnp.int32),
            pltpu.VMEM((NBUF, CHUNK, DH), jnp.float32),
            pltpu.VMEM_SHARED((NP, DH), jnp.float32),
            pltpu.VMEM_SHARED((NP, DH), jnp.float32),
            pltpu.VMEM((ZR, DH), jnp.float32),
        ] + [pltpu.SemaphoreType.DMA] * NBUF,
        interpret=interpret,
    )


_sc_segsum = _make_sc_segsum()


def _sc_degree_body(src_hbm, out_hbm, idx_s, ones_b, acc, zbuf):
    cid = lax.axis_index("c")
    sid = lax.axis_index("s")
    wid = cid * NS + sid

    pltpu.sync_copy(src_hbm.at[wid], idx_s)

    one = jnp.ones((L,), jnp.float32)
    zero = jnp.zeros((L,), jnp.float32)

    @pl.loop(0, CHUNK)
    def _fill_ones(i):
        ones_b[i, :] = one

    @pl.loop(0, RPT)
    def _zero_rows(i):
        zbuf[i, :] = zero

    pltpu.sync_copy(zbuf, acc.at[pl.ds(sid * RPT, RPT)])
    plsc.subcore_barrier()

    @pl.loop(0, CHD)
    def _chunks(ch):
        pltpu.sync_copy(ones_b, acc.at[idx_s.at[ch]], add=True)

    plsc.subcore_barrier()
    pltpu.sync_copy(acc.at[pl.ds(sid * RPT, RPT)], out_hbm.at[wid])


def _make_sc_degree(interpret=False):
    return pl.kernel(
        _sc_degree_body,
        out_type=jax.ShapeDtypeStruct((NW, RPT, DEGW), jnp.float32),
        mesh=_sc_mesh,
        compiler_params=pltpu.CompilerParams(use_tc_tiling_on_sc=False),
        scratch_types=[
            pltpu.VMEM((CHD, CHUNK), jnp.int32),
            pltpu.VMEM((CHUNK, DEGW), jnp.float32),
            pltpu.VMEM_SHARED((NP, DEGW), jnp.float32),
            pltpu.VMEM((RPT, DEGW), jnp.float32),
        ],
        interpret=interpret,
    )


_sc_degree = _make_sc_degree()


def _dinv_body(g_ref, dinv_ref):
    g = g_ref[...]
    deg = g[0, :, 0:1] + g[1, :, 0:1]
    dinv_ref[...] = jnp.where(deg > 0, lax.rsqrt(jnp.maximum(deg, 1e-12)), 0.0)


def _tc_dinv(degraw):
    return pl.pallas_call(
        _dinv_body,
        grid=(NP // BR,),
        in_specs=[pl.BlockSpec((NC, BR, DEGW), lambda i: (0, i, 0))],
        out_specs=pl.BlockSpec((BR, 1), lambda i: (i, 0)),
        out_shape=jax.ShapeDtypeStruct((NP, 1), jnp.float32),
    )(degraw)


def _dot(a, b):
    return jnp.dot(a, b, preferred_element_type=jnp.float32,
                   precision=lax.Precision.HIGHEST)


def _split(v):
    return jnp.stack([v[:, :DH], v[:, DH:]], axis=0)


def _cat(r):
    return jnp.concatenate([r[0], r[1]], axis=-1)


def _mm3(h, w_ref, b_ref, d, p_ref, y1_ref, c_ref):
    w0 = w_ref[0]
    w1 = w_ref[1]
    w2 = w_ref[2]
    p_ref[...] = _split(d * _dot(h, w2))
    y1_ref[...] = _dot(h, w1)
    c_ref[...] = _dot(h, w0 - w2) + b_ref[...]


def _k1_body(h_ref, w_ref, b_ref, dinv_ref, p_ref, y1_ref, c_ref):
    _mm3(h_ref[...], w_ref, b_ref, dinv_ref[...], p_ref, y1_ref, c_ref)


def _tc_layer_in(h, W, b, dinv, din):
    sds = jax.ShapeDtypeStruct((NP, D), jnp.float32)
    sds_s = jax.ShapeDtypeStruct((NC, NP, DH), jnp.float32)
    return pl.pallas_call(
        _k1_body,
        grid=(NP // BR,),
        in_specs=[
            pl.BlockSpec((BR, din), lambda i: (i, 0)),
            pl.BlockSpec((3, din, D), lambda i: (0, 0, 0)),
            pl.BlockSpec((1, D), lambda i: (0, 0)),
            pl.BlockSpec((BR, 1), lambda i: (i, 0)),
        ],
        out_specs=[
            pl.BlockSpec((NC, BR, DH), lambda i: (0, i, 0)),
            pl.BlockSpec((BR, D), lambda i: (i, 0)),
            pl.BlockSpec((BR, D), lambda i: (i, 0)),
        ],
        out_shape=[sds_s, sds, sds],
    )(h, W, b.reshape(1, D), dinv)


def _k13_body(c_ref, r_ref, dinv_ref, w_ref, b_ref, p_ref, y1_ref, c2_ref):
    d = dinv_ref[...]
    h = c_ref[...] - d * _cat(r_ref)
    h = jnp.maximum(h, 0.0)
    _mm3(h, w_ref, b_ref, d, p_ref, y1_ref, c2_ref)


def _tc_layer_boundary(c, rraw, dinv, W, b):
    sds = jax.ShapeDtypeStruct((NP, D), jnp.float32)
    sds_s = jax.ShapeDtypeStruct((NC, NP, DH), jnp.float32)
    return pl.pallas_call(
        _k13_body,
        grid=(NP // BR,),
        in_specs=[
            pl.BlockSpec((BR, D), lambda i: (i, 0)),
            pl.BlockSpec((NC, BR, DH), lambda i: (0, i, 0)),
            pl.BlockSpec((BR, 1), lambda i: (i, 0)),
            pl.BlockSpec((3, D, D), lambda i: (0, 0, 0)),
            pl.BlockSpec((1, D), lambda i: (0, 0)),
        ],
        out_specs=[
            pl.BlockSpec((NC, BR, DH), lambda i: (0, i, 0)),
            pl.BlockSpec((BR, D), lambda i: (i, 0)),
            pl.BlockSpec((BR, D), lambda i: (i, 0)),
        ],
        out_shape=[sds_s, sds, sds],
    )(c, rraw, dinv, W, b.reshape(1, D))


def _k2_body(z_ref, y1_ref, dinv_ref, u_ref):
    d = dinv_ref[...]
    u_ref[...] = _split(d * y1_ref[...] - (2.0 * d * d) * _cat(z_ref))


def _tc_mid(zraw, y1, dinv):
    return pl.pallas_call(
        _k2_body,
        grid=(NP // BR,),
        in_specs=[
            pl.BlockSpec((NC, BR, DH), lambda i: (0, i, 0)),
            pl.BlockSpec((BR, D), lambda i: (i, 0)),
            pl.BlockSpec((BR, 1), lambda i: (i, 0)),
        ],
        out_specs=pl.BlockSpec((NC, BR, DH), lambda i: (0, i, 0)),
        out_shape=jax.ShapeDtypeStruct((NC, NP, DH), jnp.float32),
    )(zraw, y1, dinv)


def _k3_body(c_ref, r_ref, dinv_ref, o_ref):
    o_ref[...] = c_ref[...] - dinv_ref[...] * _cat(r_ref)


def _tc_final(c, rraw, dinv):
    return pl.pallas_call(
        _k3_body,
        grid=(NP // BR,),
        in_specs=[
            pl.BlockSpec((BR, D), lambda i: (i, 0)),
            pl.BlockSpec((NC, BR, DH), lambda i: (0, i, 0)),
            pl.BlockSpec((BR, 1), lambda i: (i, 0)),
        ],
        out_specs=pl.BlockSpec((BR, D), lambda i: (i, 0)),
        out_shape=jax.ShapeDtypeStruct((NP, D), jnp.float32),
    )(c, rraw, dinv)


def _segsum(table_split, src_t, dst_t):
    raw = _sc_segsum(table_split, src_t, dst_t)
    return raw.reshape(NC, NP, DH)


def kernel(x, edge_index, W1, b1, W2, b2, W3, b3):
    xp = jnp.zeros((NP, D_IN), jnp.float32).at[:N].set(x)
    pad = jnp.full((EP - E,), PAD_IDX, jnp.int32)
    src_all = jnp.concatenate([edge_index[0], pad])
    dst_all = jnp.concatenate([edge_index[1], pad])
    src_t = src_all.reshape(NS, CH, CHUNK)
    dst_t = dst_all.reshape(NS, CH, CHUNK)
    src_deg = src_all.reshape(NW, CHD, CHUNK)

    degraw = _sc_degree(src_deg).reshape(NC, NP, DEGW)
    dinv = _tc_dinv(degraw)

    p, y1, c = _tc_layer_in(xp, W1, b1, dinv, D_IN)
    for (W, b) in ((W2, b2), (W3, b3)):
        zraw = _segsum(p, src_t, dst_t)
        u = _tc_mid(zraw, y1, dinv)
        rraw = _segsum(u, src_t, dst_t)
        p, y1, c = _tc_layer_boundary(c, rraw, dinv, W, b)
    zraw = _segsum(p, src_t, dst_t)
    u = _tc_mid(zraw, y1, dinv)
    rraw = _segsum(u, src_t, dst_t)
    out = _tc_final(c, rraw, dinv)
    return out[:N]


# speedup vs baseline: 2.4079x; 1.1081x over previous
import functools

import jax
import jax.numpy as jnp
from jax import lax
from jax.experimental import pallas as pl
from jax.experimental.pallas import tpu as pltpu
from jax.experimental.pallas import tpu_sc as plsc

N = 10000
E = 320000
D_IN = 128
D = 64
DH = D // 2

NC = 2
NS = 16
NW = NC * NS
L = 16

NP = 10240
RPT = NP // NS
CHUNK = 128
CH = 160
EP = NS * CH * CHUNK
PAD_IDX = NP - 1
DEGW = 16
CHD = 80

BR = 256

_sc_mesh = plsc.VectorSubcoreMesh(
    core_axis_name="c", subcore_axis_name="s", num_cores=NC, num_subcores=NS
)


NBUF = 4
ZR = 128


def _sc_layer_body(p_hbm, yp_hbm, bb_hbm, src_hbm, dst_hbm, out_hbm,
                   idx_s, idx_d, rows, tbl, acc, zbuf, zb, yb, bbuf, ub,
                   *sems):
    cid = lax.axis_index("c")
    sid = lax.axis_index("s")
    wid = cid * NS + sid
    base = sid * RPT

    pltpu.sync_copy(p_hbm.at[cid, pl.ds(base, RPT)], tbl.at[pl.ds(base, RPT)])

    pltpu.sync_copy(src_hbm.at[sid], idx_s)
    pltpu.sync_copy(dst_hbm.at[sid], idx_d)

    zero = jnp.zeros((L,), jnp.float32)

    @pl.loop(0, ZR)
    def _zero_rows(i):
        for j in range(DH // L):
            zbuf[i, pl.ds(j * L, L)] = zero

    for r in range(RPT // ZR):
        pltpu.sync_copy(zbuf, acc.at[pl.ds(base + r * ZR, ZR)])
    plsc.subcore_barrier()

    def _segsum_ring():
        for b in range(NBUF):
            pltpu.async_copy(tbl.at[idx_s.at[b]], rows.at[b], sems[b])

        @pl.loop(0, CH, step=NBUF)
        def _chunks(ch):
            for b in range(NBUF):
                pltpu.make_async_copy(
                    tbl.at[idx_s.at[ch + b]], rows.at[b], sems[b]).wait()
                pltpu.sync_copy(rows.at[b], acc.at[idx_d.at[ch + b]], add=True)

                @pl.when(ch + NBUF + b < CH)
                def _():
                    pltpu.async_copy(
                        tbl.at[idx_s.at[ch + NBUF + b]], rows.at[b], sems[b])

    _segsum_ring()
    plsc.subcore_barrier()

    for r in range(RPT // ZR):
        off = base + r * ZR
        pltpu.sync_copy(acc.at[pl.ds(off, ZR)], zb)
        pltpu.sync_copy(yp_hbm.at[cid, pl.ds(off, ZR)], yb)
        pltpu.sync_copy(bb_hbm.at[cid, pl.ds(off, ZR)], bbuf)

        @pl.loop(0, ZR)
        def _mid_rows(i):
            for j in range(DH // L):
                s = pl.ds(j * L, L)
                ub[i, s] = yb[i, s] - bbuf[i, s] * zb[i, s]

        pltpu.sync_copy(ub, tbl.at[pl.ds(off, ZR)])
        pltpu.sync_copy(zbuf, acc.at[pl.ds(off, ZR)])
    plsc.subcore_barrier()

    _segsum_ring()
    plsc.subcore_barrier()

    pltpu.sync_copy(acc.at[pl.ds(base, RPT)], out_hbm.at[wid])


def _make_sc_layer(interpret=False):
    return pl.kernel(
        _sc_layer_body,
        out_type=jax.ShapeDtypeStruct((NW, RPT, DH), jnp.float32),
        mesh=_sc_mesh,
        compiler_params=pltpu.CompilerParams(use_tc_tiling_on_sc=False),
        scratch_types=[
            pltpu.VMEM((CH, CHUNK), jnp.int32),
            pltpu.VMEM((CH, CHUNK), jnp.int32),
            pltpu.VMEM((NBUF, CHUNK, DH), jnp.float32),
            pltpu.VMEM_SHARED((NP, DH), jnp.float32),
            pltpu.VMEM_SHARED((NP, DH), jnp.float32),
            pltpu.VMEM((ZR, DH), jnp.float32),
            pltpu.VMEM((ZR, DH), jnp.float32),
            pltpu.VMEM((ZR, DH), jnp.float32),
            pltpu.VMEM((ZR, DH), jnp.float32),
            pltpu.VMEM((ZR, DH), jnp.float32),
        ] + [pltpu.SemaphoreType.DMA] * NBUF,
        interpret=interpret,
    )


_sc_layer = _make_sc_layer()


def _sc_degree_body(src_hbm, out_hbm, idx_s, ones_b, acc, zbuf):
    cid = lax.axis_index("c")
    sid = lax.axis_index("s")
    wid = cid * NS + sid

    pltpu.sync_copy(src_hbm.at[wid], idx_s)

    one = jnp.ones((L,), jnp.float32)
    zero = jnp.zeros((L,), jnp.float32)

    @pl.loop(0, CHUNK)
    def _fill_ones(i):
        ones_b[i, :] = one

    @pl.loop(0, RPT)
    def _zero_rows(i):
        zbuf[i, :] = zero

    pltpu.sync_copy(zbuf, acc.at[pl.ds(sid * RPT, RPT)])
    plsc.subcore_barrier()

    @pl.loop(0, CHD)
    def _chunks(ch):
        pltpu.sync_copy(ones_b, acc.at[idx_s.at[ch]], add=True)

    plsc.subcore_barrier()
    pltpu.sync_copy(acc.at[pl.ds(sid * RPT, RPT)], out_hbm.at[wid])


def _make_sc_degree(interpret=False):
    return pl.kernel(
        _sc_degree_body,
        out_type=jax.ShapeDtypeStruct((NW, RPT, DEGW), jnp.float32),
        mesh=_sc_mesh,
        compiler_params=pltpu.CompilerParams(use_tc_tiling_on_sc=False),
        scratch_types=[
            pltpu.VMEM((CHD, CHUNK), jnp.int32),
            pltpu.VMEM((CHUNK, DEGW), jnp.float32),
            pltpu.VMEM_SHARED((NP, DEGW), jnp.float32),
            pltpu.VMEM((RPT, DEGW), jnp.float32),
        ],
        interpret=interpret,
    )


_sc_degree = _make_sc_degree()


def _dinv_body(g_ref, dinv_ref, bb_ref):
    g = g_ref[...]
    deg = g[0, :, 0:1] + g[1, :, 0:1]
    d = jnp.where(deg > 0, lax.rsqrt(jnp.maximum(deg, 1e-12)), 0.0)
    dinv_ref[...] = d
    bb = jnp.broadcast_to(2.0 * d * d, (BR, DH))
    bb_ref[...] = jnp.stack([bb, bb], axis=0)


def _tc_dinv(degraw):
    return pl.pallas_call(
        _dinv_body,
        grid=(NP // BR,),
        in_specs=[pl.BlockSpec((NC, BR, DEGW), lambda i: (0, i, 0))],
        out_specs=[
            pl.BlockSpec((BR, 1), lambda i: (i, 0)),
            pl.BlockSpec((NC, BR, DH), lambda i: (0, i, 0)),
        ],
        out_shape=[
            jax.ShapeDtypeStruct((NP, 1), jnp.float32),
            jax.ShapeDtypeStruct((NC, NP, DH), jnp.float32),
        ],
    )(degraw)


def _dot(a, b):
    return jnp.dot(a, b, preferred_element_type=jnp.float32,
                   precision=lax.Precision.HIGHEST)


def _split(v):
    return jnp.stack([v[:, :DH], v[:, DH:]], axis=0)


def _cat(r):
    return jnp.concatenate([r[0], r[1]], axis=-1)


def _mm3(h, w_ref, b_ref, d, p_ref, yp_ref, c_ref):
    w0 = w_ref[0]
    w1 = w_ref[1]
    w2 = w_ref[2]
    p_ref[...] = _split(d * _dot(h, w2))
    yp_ref[...] = _split(d * _dot(h, w1))
    c_ref[...] = _dot(h, w0 - w2) + b_ref[...]


def _k1_body(h_ref, w_ref, b_ref, dinv_ref, p_ref, yp_ref, c_ref):
    _mm3(h_ref[...], w_ref, b_ref, dinv_ref[...], p_ref, yp_ref, c_ref)


def _tc_layer_in(h, W, b, dinv, din):
    sds = jax.ShapeDtypeStruct((NP, D), jnp.float32)
    sds_s = jax.ShapeDtypeStruct((NC, NP, DH), jnp.float32)
    return pl.pallas_call(
        _k1_body,
        grid=(NP // BR,),
        in_specs=[
            pl.BlockSpec((BR, din), lambda i: (i, 0)),
            pl.BlockSpec((3, din, D), lambda i: (0, 0, 0)),
            pl.BlockSpec((1, D), lambda i: (0, 0)),
            pl.BlockSpec((BR, 1), lambda i: (i, 0)),
        ],
        out_specs=[
            pl.BlockSpec((NC, BR, DH), lambda i: (0, i, 0)),
            pl.BlockSpec((NC, BR, DH), lambda i: (0, i, 0)),
            pl.BlockSpec((BR, D), lambda i: (i, 0)),
        ],
        out_shape=[sds_s, sds_s, sds],
    )(h, W, b.reshape(1, D), dinv)


def _k13_body(c_ref, r_ref, dinv_ref, w_ref, b_ref, p_ref, yp_ref, c2_ref):
    d = dinv_ref[...]
    h = c_ref[...] - d * _cat(r_ref)
    h = jnp.maximum(h, 0.0)
    _mm3(h, w_ref, b_ref, d, p_ref, yp_ref, c2_ref)


def _tc_layer_boundary(c, rraw, dinv, W, b):
    sds = jax.ShapeDtypeStruct((NP, D), jnp.float32)
    sds_s = jax.ShapeDtypeStruct((NC, NP, DH), jnp.float32)
    return pl.pallas_call(
        _k13_body,
        grid=(NP // BR,),
        in_specs=[
            pl.BlockSpec((BR, D), lambda i: (i, 0)),
            pl.BlockSpec((NC, BR, DH), lambda i: (0, i, 0)),
            pl.BlockSpec((BR, 1), lambda i: (i, 0)),
            pl.BlockSpec((3, D, D), lambda i: (0, 0, 0)),
            pl.BlockSpec((1, D), lambda i: (0, 0)),
        ],
        out_specs=[
            pl.BlockSpec((NC, BR, DH), lambda i: (0, i, 0)),
            pl.BlockSpec((NC, BR, DH), lambda i: (0, i, 0)),
            pl.BlockSpec((BR, D), lambda i: (i, 0)),
        ],
        out_shape=[sds_s, sds_s, sds],
    )(c, rraw, dinv, W, b.reshape(1, D))


def _k3_body(c_ref, r_ref, dinv_ref, o_ref):
    o_ref[...] = c_ref[...] - dinv_ref[...] * _cat(r_ref)


def _tc_final(c, rraw, dinv):
    return pl.pallas_call(
        _k3_body,
        grid=(NP // BR,),
        in_specs=[
            pl.BlockSpec((BR, D), lambda i: (i, 0)),
            pl.BlockSpec((NC, BR, DH), lambda i: (0, i, 0)),
            pl.BlockSpec((BR, 1), lambda i: (i, 0)),
        ],
        out_specs=pl.BlockSpec((BR, D), lambda i: (i, 0)),
        out_shape=jax.ShapeDtypeStruct((NP, D), jnp.float32),
    )(c, rraw, dinv)


def kernel(x, edge_index, W1, b1, W2, b2, W3, b3):
    xp = jnp.zeros((NP, D_IN), jnp.float32).at[:N].set(x)
    pad = jnp.full((EP - E,), PAD_IDX, jnp.int32)
    src_all = jnp.concatenate([edge_index[0], pad])
    dst_all = jnp.concatenate([edge_index[1], pad])
    src_t = src_all.reshape(NS, CH, CHUNK)
    dst_t = dst_all.reshape(NS, CH, CHUNK)
    src_deg = src_all.reshape(NW, CHD, CHUNK)

    degraw = _sc_degree(src_deg).reshape(NC, NP, DEGW)
    dinv, bb = _tc_dinv(degraw)

    p, yp, c = _tc_layer_in(xp, W1, b1, dinv, D_IN)
    for (W, b) in ((W2, b2), (W3, b3)):
        rraw = _sc_layer(p, yp, bb, src_t, dst_t).reshape(NC, NP, DH)
        p, yp, c = _tc_layer_boundary(c, rraw, dinv, W, b)
    rraw = _sc_layer(p, yp, bb, src_t, dst_t).reshape(NC, NP, DH)
    return _tc_final(c, rraw, dinv)[:N]
